# Initial kernel scaffold; baseline (speedup 1.0000x reference)
import jax
import jax.numpy as jnp
from jax.experimental import pallas as pl


def kernel(query, keys, values):
    raise NotImplementedError("write your pallas kernel here")


# speedup vs baseline: 1.3325x; 1.3325x over previous
import functools

import jax
import jax.numpy as jnp
from jax import lax
from jax.experimental import pallas as pl
from jax.experimental.pallas import tpu as pltpu
from jax.experimental.pallas import tpu_sc as plsc

Q = 1024
D = 128
N = 100000
BN = 2000
NBLK = N // BN


def _argmax_body(q_ref, k_ref, idx_ref, bestv_ref, besti_ref):
    j = pl.program_id(0)

    @pl.when(j == 0)
    def _init():
        bestv_ref[...] = jnp.full_like(bestv_ref, -jnp.inf)
        besti_ref[...] = jnp.zeros_like(besti_ref)

    kb = k_ref[...]
    kn2 = jnp.sum(kb * kb, axis=1, keepdims=True)
    inv = jnp.where(kn2 > 0, 1.0 / jnp.sqrt(kn2), 0.0)
    dots = lax.dot_general(
        kb, q_ref[...],
        dimension_numbers=(((1,), (1,)), ((), ())),
        preferred_element_type=jnp.float32,
        precision=lax.Precision.DEFAULT,
    )
    sims = dots * inv
    blk_max = jnp.max(sims, axis=0, keepdims=True)
    row = lax.broadcasted_iota(jnp.int32, sims.shape, 0)
    cand = jnp.where(sims == blk_max, row, N)
    blk_arg = jnp.min(cand, axis=0, keepdims=True) + j * BN

    better = blk_max > bestv_ref[...]
    bestv_ref[...] = jnp.where(better, blk_max, bestv_ref[...])
    besti_ref[...] = jnp.where(better, blk_arg, besti_ref[...])

    @pl.when(j == NBLK - 1)
    def _done():
        idx_ref[...] = besti_ref[0, :]


def _argmax_call(query, keys):
    return pl.pallas_call(
        _argmax_body,
        grid=(NBLK,),
        in_specs=[
            pl.BlockSpec((Q, D), lambda j: (0, 0)),
            pl.BlockSpec((BN, D), lambda j: (j, 0)),
        ],
        out_specs=pl.BlockSpec((Q,), lambda j: (0,)),
        out_shape=jax.ShapeDtypeStruct((Q,), jnp.int32),
        scratch_shapes=[
            pltpu.VMEM((1, Q), jnp.float32),
            pltpu.VMEM((1, Q), jnp.int32),
        ],
        compiler_params=pltpu.CompilerParams(
            dimension_semantics=("arbitrary",),
        ),
    )(query, keys)


_NC = 2
_NS = 16
_NW = _NC * _NS
_BPW = Q // _NW


@functools.lru_cache(maxsize=1)
def _make_gather_rows():
    @functools.partial(
        pl.kernel,
        mesh=plsc.VectorSubcoreMesh(
            core_axis_name="c", subcore_axis_name="s",
            num_cores=_NC, num_subcores=_NS),
        out_type=jax.ShapeDtypeStruct((Q, D), jnp.float32),
        scratch_types=[
            pltpu.VMEM((_BPW,), jnp.int32),
            pltpu.VMEM((_BPW, D), jnp.float32),
            pltpu.SemaphoreType.DMA,
        ],
    )
    def _gather_rows(values_hbm, idx_hbm, out_hbm, idx_v, rows_v, sem):
        wid = lax.axis_index("s") * _NC + lax.axis_index("c")
        base = wid * _BPW
        pltpu.sync_copy(idx_hbm.at[pl.ds(base, _BPW)], idx_v)
        pltpu.async_copy(values_hbm.at[idx_v], rows_v, sem).wait()
        pltpu.sync_copy(rows_v, out_hbm.at[pl.ds(base, _BPW)])

    return _gather_rows


def kernel(query, keys, values):
    idx = _argmax_call(query, keys)
    return _make_gather_rows()(values, idx)


# speedup vs baseline: 1.8850x; 1.4146x over previous
import functools

import jax
import jax.numpy as jnp
from jax import lax
from jax.experimental import pallas as pl
from jax.experimental.pallas import tpu as pltpu
from jax.experimental.pallas import tpu_sc as plsc

Q = 1024
D = 128
N = 100000
BN = 2000
NBLK = N // BN
T = BN // 8
TT = T // 2


def _merge_tiesafe(rv, ri, cv, ci):
    take = (cv > rv) | ((cv == rv) & (ci < ri))
    return jnp.where(take, cv, rv), jnp.where(take, ci, ri)


def _argmax_body(q_ref, k_ref, idx_ref, bestv_ref, besti_ref):
    j = pl.program_id(0)

    @pl.when(j == 0)
    def _init():
        bestv_ref[...] = jnp.full_like(bestv_ref, -jnp.inf)
        besti_ref[...] = jnp.zeros_like(besti_ref)

    kb = k_ref[...]
    kn2 = jnp.sum(kb * kb, axis=1, keepdims=True)
    inv = jnp.where(kn2 > 0, 1.0 / jnp.sqrt(kn2), 0.0)
    dots = lax.dot_general(
        kb, q_ref[...],
        dimension_numbers=(((1,), (1,)), ((), ())),
        preferred_element_type=jnp.float32,
        precision=lax.Precision.DEFAULT,
    )
    sims = dots * inv

    pairs = sims.reshape(TT, 2, 8, Q)
    a = pairs[:, 0]
    b = pairs[:, 1]
    gt = b > a
    tidx = lax.broadcasted_iota(jnp.int32, (TT, 8, Q), 0) * 2
    idx = jnp.where(gt, tidx + 1, tidx)
    val = jnp.maximum(a, b)
    carries = []
    t = TT
    while t > 1:
        if t % 2:
            carries.append((val[t - 1 :], idx[t - 1 :]))
            val, idx = val[: t - 1], idx[: t - 1]
            t -= 1
        vp = val.reshape(t // 2, 2, 8, Q)
        ip = idx.reshape(t // 2, 2, 8, Q)
        av, ai = vp[:, 0], ip[:, 0]
        bv, bi = vp[:, 1], ip[:, 1]
        gt = bv > av
        val = jnp.maximum(av, bv)
        idx = jnp.where(gt, bi, ai)
        t //= 2
    rv, ri = val[0], idx[0]
    for cv, ci in reversed(carries):
        rv, ri = _merge_tiesafe(rv, ri, cv[0], ci[0])

    better = rv > bestv_ref[...]
    bestv_ref[...] = jnp.where(better, rv, bestv_ref[...])
    besti_ref[...] = jnp.where(better, ri + j * T, besti_ref[...])

    @pl.when(j == NBLK - 1)
    def _done():
        bv = bestv_ref[...]
        rows = besti_ref[...] * 8 + lax.broadcasted_iota(jnp.int32, (8, Q), 0)
        m = jnp.max(bv, axis=0, keepdims=True)
        cand = jnp.where(bv == m, rows, jnp.int32(0x7FFFFFFF))
        idx_ref[...] = jnp.min(cand, axis=0)


def _argmax_call(query, keys):
    return pl.pallas_call(
        _argmax_body,
        grid=(NBLK,),
        in_specs=[
            pl.BlockSpec((Q, D), lambda j: (0, 0)),
            pl.BlockSpec((BN, D), lambda j: (j, 0)),
        ],
        out_specs=pl.BlockSpec((Q,), lambda j: (0,)),
        out_shape=jax.ShapeDtypeStruct((Q,), jnp.int32),
        scratch_shapes=[
            pltpu.VMEM((8, Q), jnp.float32),
            pltpu.VMEM((8, Q), jnp.int32),
        ],
        compiler_params=pltpu.CompilerParams(
            dimension_semantics=("arbitrary",),
        ),
    )(query, keys)


_NC = 2
_NS = 16
_NW = _NC * _NS
_BPW = Q // _NW


@functools.lru_cache(maxsize=1)
def _make_gather_rows():
    @functools.partial(
        pl.kernel,
        mesh=plsc.VectorSubcoreMesh(
            core_axis_name="c", subcore_axis_name="s",
            num_cores=_NC, num_subcores=_NS),
        out_type=jax.ShapeDtypeStruct((Q, D), jnp.float32),
        scratch_types=[
            pltpu.VMEM((_BPW,), jnp.int32),
            pltpu.VMEM((_BPW, D), jnp.float32),
            pltpu.SemaphoreType.DMA,
        ],
    )
    def _gather_rows(values_hbm, idx_hbm, out_hbm, idx_v, rows_v, sem):
        wid = lax.axis_index("s") * _NC + lax.axis_index("c")
        base = wid * _BPW
        pltpu.sync_copy(idx_hbm.at[pl.ds(base, _BPW)], idx_v)
        pltpu.async_copy(values_hbm.at[idx_v], rows_v, sem).wait()
        pltpu.sync_copy(rows_v, out_hbm.at[pl.ds(base, _BPW)])

    return _gather_rows


def kernel(query, keys, values):
    idx = _argmax_call(query, keys)
    return _make_gather_rows()(values, idx)


# speedup vs baseline: 1.9966x; 1.0592x over previous
import functools

import jax
import jax.numpy as jnp
from jax import lax
from jax.experimental import pallas as pl
from jax.experimental.pallas import tpu as pltpu
from jax.experimental.pallas import tpu_sc as plsc

Q = 1024
D = 128
N = 100000
BN = 4000
NBLK = N // BN
T = BN // 8
TT = T // 2


def _merge_tiesafe(rv, ri, cv, ci):
    take = (cv > rv) | ((cv == rv) & (ci < ri))
    return jnp.where(take, cv, rv), jnp.where(take, ci, ri)


def _argmax_body(q_ref, k_ref, idx_ref, bestv_ref, besti_ref):
    j = pl.program_id(0)

    @pl.when(j == 0)
    def _init():
        bestv_ref[...] = jnp.full_like(bestv_ref, -jnp.inf)
        besti_ref[...] = jnp.zeros_like(besti_ref)

    kb = k_ref[...]
    kn2 = jnp.sum(kb * kb, axis=1, keepdims=True)
    inv = jnp.where(kn2 > 0, 1.0 / jnp.sqrt(kn2), 0.0)
    dots = lax.dot_general(
        kb, q_ref[...],
        dimension_numbers=(((1,), (1,)), ((), ())),
        preferred_element_type=jnp.float32,
        precision=lax.Precision.DEFAULT,
    )
    sims = dots * inv

    pairs = sims.reshape(TT, 2, 8, Q)
    a = pairs[:, 0]
    b = pairs[:, 1]
    gt = b > a
    tidx = lax.broadcasted_iota(jnp.int32, (TT, 8, Q), 0) * 2
    idx = jnp.where(gt, tidx + 1, tidx)
    val = jnp.maximum(a, b)
    carries = []
    t = TT
    while t > 1:
        if t % 2:
            carries.append((val[t - 1 :], idx[t - 1 :]))
            val, idx = val[: t - 1], idx[: t - 1]
            t -= 1
        vp = val.reshape(t // 2, 2, 8, Q)
        ip = idx.reshape(t // 2, 2, 8, Q)
        av, ai = vp[:, 0], ip[:, 0]
        bv, bi = vp[:, 1], ip[:, 1]
        gt = bv > av
        val = jnp.maximum(av, bv)
        idx = jnp.where(gt, bi, ai)
        t //= 2
    rv, ri = val[0], idx[0]
    for cv, ci in reversed(carries):
        rv, ri = _merge_tiesafe(rv, ri, cv[0], ci[0])

    better = rv > bestv_ref[...]
    bestv_ref[...] = jnp.where(better, rv, bestv_ref[...])
    besti_ref[...] = jnp.where(better, ri + j * T, besti_ref[...])

    @pl.when(j == NBLK - 1)
    def _done():
        bv = bestv_ref[...]
        rows = besti_ref[...] * 8 + lax.broadcasted_iota(jnp.int32, (8, Q), 0)
        m = jnp.max(bv, axis=0, keepdims=True)
        cand = jnp.where(bv == m, rows, jnp.int32(0x7FFFFFFF))
        idx_ref[...] = jnp.min(cand, axis=0)


def _argmax_call(query, keys):
    return pl.pallas_call(
        _argmax_body,
        grid=(NBLK,),
        in_specs=[
            pl.BlockSpec((Q, D), lambda j: (0, 0)),
            pl.BlockSpec((BN, D), lambda j: (j, 0)),
        ],
        out_specs=pl.BlockSpec((Q,), lambda j: (0,)),
        out_shape=jax.ShapeDtypeStruct((Q,), jnp.int32),
        scratch_shapes=[
            pltpu.VMEM((8, Q), jnp.float32),
            pltpu.VMEM((8, Q), jnp.int32),
        ],
        compiler_params=pltpu.CompilerParams(
            dimension_semantics=("arbitrary",),
        ),
    )(query, keys)


_NC = 2
_NS = 16
_NW = _NC * _NS
_BPW = Q // _NW


@functools.lru_cache(maxsize=1)
def _make_gather_rows():
    @functools.partial(
        pl.kernel,
        mesh=plsc.VectorSubcoreMesh(
            core_axis_name="c", subcore_axis_name="s",
            num_cores=_NC, num_subcores=_NS),
        out_type=jax.ShapeDtypeStruct((Q, D), jnp.float32),
        scratch_types=[
            pltpu.VMEM((_BPW,), jnp.int32),
            pltpu.VMEM((_BPW, D), jnp.float32),
            pltpu.SemaphoreType.DMA,
        ],
    )
    def _gather_rows(values_hbm, idx_hbm, out_hbm, idx_v, rows_v, sem):
        wid = lax.axis_index("s") * _NC + lax.axis_index("c")
        base = wid * _BPW
        pltpu.sync_copy(idx_hbm.at[pl.ds(base, _BPW)], idx_v)
        pltpu.async_copy(values_hbm.at[idx_v], rows_v, sem).wait()
        pltpu.sync_copy(rows_v, out_hbm.at[pl.ds(base, _BPW)])

    return _gather_rows


def kernel(query, keys, values):
    idx = _argmax_call(query, keys)
    return _make_gather_rows()(values, idx)
